# Initial kernel scaffold; baseline (speedup 1.0000x reference)
import jax
import jax.numpy as jnp
from jax.experimental import pallas as pl


def kernel(x, edge_index, W_l1, W_r1, b1, W_l2, W_r2, b2):
    raise NotImplementedError("write your pallas kernel here")


# speedup vs baseline: 5.8617x; 5.8617x over previous
import functools

import jax
import jax.numpy as jnp
from jax import lax
from jax.experimental import pallas as pl
from jax.experimental.pallas import tpu as pltpu
from jax.experimental.pallas import tpu_sc as plsc

N = 10000
D = 128
HD = D // 2
CW = 16
NC, NS = 2, 16
CHUNK = 125
NB = 16
BL = 1000
R0 = (N // NS) // 8 * 8
TAIL = N - NS * R0


def _sc_mesh():
    return plsc.VectorSubcoreMesh(
        core_axis_name="c", subcore_axis_name="s",
        num_cores=NC, num_subcores=NS)


def _make_sc_agg(n_edges, with_counts):
    ept = n_edges // NS
    nch = ept // CHUNK
    nblk = nch // NB

    out_type = [jax.ShapeDtypeStruct((NC, N, HD), jnp.float32)]
    scratch = [
        pltpu.VMEM((NB, CHUNK), jnp.int32),
        pltpu.VMEM((NB, CHUNK), jnp.int32),
        pltpu.VMEM((CHUNK, HD), jnp.float32),
        pltpu.VMEM_SHARED((N, HD), jnp.float32),
        pltpu.SemaphoreType.DMA,
    ]
    if with_counts:
        out_type.append(jax.ShapeDtypeStruct((NC * N, CW), jnp.float32))
        scratch += [
            pltpu.VMEM((CHUNK, CW), jnp.float32),
            pltpu.VMEM_SHARED((N, CW), jnp.float32),
        ]

    @functools.partial(pl.kernel,
                       out_type=tuple(out_type) if with_counts else out_type[0],
                       mesh=_sc_mesh(), scratch_types=scratch,
                       compiler_params=pltpu.CompilerParams(
                           use_tc_tiling_on_sc=False))
    def k(*refs):
        if with_counts:
            (table3, src4, dst4, zf, zc, ones_in, outf3, outc,
             srcv, dstv, rows, accf, sem, onesv, accc) = refs
        else:
            (table3, src4, dst4, zf, outf3,
             srcv, dstv, rows, accf, sem) = refs
        c = lax.axis_index("c")
        s = lax.axis_index("s")

        pltpu.sync_copy(zf.at[pl.ds(s * R0, R0)], accf.at[pl.ds(s * R0, R0)])
        if with_counts:
            pltpu.sync_copy(zc.at[pl.ds(s * R0, R0)],
                            accc.at[pl.ds(s * R0, R0)])
            pltpu.sync_copy(ones_in, onesv)

        @pl.when(s == NS - 1)
        def _():
            pltpu.sync_copy(zf.at[pl.ds(NS * R0, TAIL)],
                            accf.at[pl.ds(NS * R0, TAIL)])
            if with_counts:
                pltpu.sync_copy(zc.at[pl.ds(NS * R0, TAIL)],
                                accc.at[pl.ds(NS * R0, TAIL)])

        plsc.subcore_barrier()

        def blk(b, carry):
            pltpu.sync_copy(src4.at[s, b], srcv)
            pltpu.sync_copy(dst4.at[s, b], dstv)

            def step(j, carry2):
                pltpu.async_copy(table3.at[c].at[srcv.at[j]], rows,
                                 sem).wait()
                pltpu.sync_copy(rows, accf.at[dstv.at[j]], add=True)
                if with_counts:
                    @pl.when(((b * NB + j + c) % 2) == 0)
                    def _():
                        pltpu.sync_copy(onesv, accc.at[dstv.at[j]],
                                        add=True)
                return carry2

            lax.fori_loop(0, NB, step, 0)
            return carry

        lax.fori_loop(0, nblk, blk, 0)
        plsc.subcore_barrier()

        pltpu.sync_copy(accf.at[pl.ds(s * R0, R0)],
                        outf3.at[c].at[pl.ds(s * R0, R0)])
        if with_counts:
            pltpu.sync_copy(accc.at[pl.ds(s * R0, R0)],
                            outc.at[pl.ds(c * N + s * R0, R0)])

        @pl.when(s == NS - 1)
        def _():
            pltpu.sync_copy(accf.at[pl.ds(NS * R0, TAIL)],
                            outf3.at[c].at[pl.ds(NS * R0, TAIL)])
            if with_counts:
                pltpu.sync_copy(accc.at[pl.ds(NS * R0, TAIL)],
                                outc.at[pl.ds(c * N + NS * R0, TAIL)])

    return k


def _tc1_body(a3, c0, c1, xr, wl1a, wl1b, wr1, b1r, wl2a, wl2b, wr2, b2r,
              g3_ref, r2_ref):
    cnt = c0[...] + c1[...]
    rinv = 1.0 / jnp.maximum(cnt[:, 0:1], 1.0)
    h = jnp.dot(a3[0] * rinv, wl1a[...], preferred_element_type=jnp.float32)
    h += jnp.dot(a3[1] * rinv, wl1b[...], preferred_element_type=jnp.float32)
    h += jnp.dot(xr[...], wr1[...], preferred_element_type=jnp.float32)
    h = jnp.maximum(h + b1r[...], 0.0)
    g3_ref[0] = jnp.dot(h, wl2a[...], preferred_element_type=jnp.float32)
    g3_ref[1] = jnp.dot(h, wl2b[...], preferred_element_type=jnp.float32)
    r2_ref[...] = (jnp.dot(h, wr2[...], preferred_element_type=jnp.float32)
                   + b2r[...])


def _tc2_body(gs3, c0, c1, r2r, out_ref):
    cnt = c0[...] + c1[...]
    rinv = 1.0 / jnp.maximum(cnt[:, 0:1], 1.0)
    out_ref[:, 0:HD] = gs3[0] * rinv + r2r[:, 0:HD]
    out_ref[:, HD:D] = gs3[1] * rinv + r2r[:, HD:D]


def _half_spec():
    return pl.BlockSpec((2, BL, HD), lambda i: (0, i, 0))


def _cnt_spec(half):
    off = half * (N // BL)
    return pl.BlockSpec((BL, CW), lambda i, o=off: (i + o, 0))


def _full_spec(shape):
    n = len(shape)
    return pl.BlockSpec(shape, lambda i: (0,) * n)


def _row_spec():
    return pl.BlockSpec((BL, D), lambda i: (i, 0))


def kernel(x, edge_index, W_l1, W_r1, b1, W_l2, W_r2, b2):
    n_edges = edge_index.shape[1]
    nch = n_edges // NS // CHUNK
    nblk = nch // NB
    src4 = edge_index[0].reshape(NS, nblk, NB, CHUNK)
    dst4 = edge_index[1].reshape(NS, nblk, NB, CHUNK)
    x3 = x.reshape(N, 2, HD).transpose(1, 0, 2)
    zf = jnp.zeros((N, HD), jnp.float32)
    zc = jnp.zeros((N, CW), jnp.float32)
    ones_in = jnp.ones((CHUNK, CW), jnp.float32)
    b1r = b1.reshape(1, D)
    b2r = b2.reshape(1, D)
    wl1a, wl1b = W_l1[:HD], W_l1[HD:]
    wl2a, wl2b = W_l2[:, :HD], W_l2[:, HD:]

    agg3, cnts = _make_sc_agg(n_edges, True)(x3, src4, dst4, zf, zc, ones_in)

    grid = (N // BL,)
    g3, r2 = pl.pallas_call(
        _tc1_body,
        grid=grid,
        in_specs=[
            _half_spec(), _cnt_spec(0), _cnt_spec(1),
            _row_spec(),
            _full_spec((HD, D)), _full_spec((HD, D)), _full_spec((D, D)),
            _full_spec((1, D)),
            _full_spec((D, HD)), _full_spec((D, HD)), _full_spec((D, D)),
            _full_spec((1, D)),
        ],
        out_specs=[_half_spec(), _row_spec()],
        out_shape=[jax.ShapeDtypeStruct((2, N, HD), jnp.float32),
                   jax.ShapeDtypeStruct((N, D), jnp.float32)],
    )(agg3, cnts, cnts, x, wl1a, wl1b, W_r1, b1r, wl2a, wl2b, W_r2, b2r)

    gs3 = _make_sc_agg(n_edges, False)(g3, src4, dst4, zf)

    out = pl.pallas_call(
        _tc2_body,
        grid=grid,
        in_specs=[_half_spec(), _cnt_spec(0), _cnt_spec(1), _row_spec()],
        out_specs=_row_spec(),
        out_shape=jax.ShapeDtypeStruct((N, D), jnp.float32),
    )(gs3, cnts, cnts, r2)

    return out


# speedup vs baseline: 9.5501x; 1.6292x over previous
import functools

import jax
import jax.numpy as jnp
from jax import lax
from jax.experimental import pallas as pl
from jax.experimental.pallas import tpu as pltpu
from jax.experimental.pallas import tpu_sc as plsc

N = 10000
D = 128
HD = D // 2
CW = 16
NC, NS = 2, 16
CHUNK = 125
NB = 16
BL = 1000
R0 = (N // NS) // 8 * 8
TAIL = N - NS * R0


def _sc_mesh():
    return plsc.VectorSubcoreMesh(
        core_axis_name="c", subcore_axis_name="s",
        num_cores=NC, num_subcores=NS)


def _make_sc_agg(n_edges, with_counts):
    ept = n_edges // NS
    nch = ept // CHUNK
    nblk = nch // NB
    nbuf = 4

    out_type = [jax.ShapeDtypeStruct((NC, N, HD), jnp.float32)]
    scratch = [
        pltpu.VMEM((NB, CHUNK), jnp.int32),
        pltpu.VMEM((NB, CHUNK), jnp.int32),
        [pltpu.VMEM((CHUNK, HD), jnp.float32) for _ in range(nbuf)],
        [pltpu.SemaphoreType.DMA for _ in range(nbuf)],
        [pltpu.SemaphoreType.DMA for _ in range(nbuf)],
        pltpu.VMEM_SHARED((N, HD), jnp.float32),
        pltpu.SemaphoreType.DMA,
    ]
    if with_counts:
        out_type.append(jax.ShapeDtypeStruct((NC * N, CW), jnp.float32))
        scratch += [
            pltpu.VMEM((CHUNK, CW), jnp.float32),
            pltpu.VMEM_SHARED((N, CW), jnp.float32),
        ]

    @functools.partial(pl.kernel,
                       out_type=tuple(out_type) if with_counts else out_type[0],
                       mesh=_sc_mesh(), scratch_types=scratch,
                       compiler_params=pltpu.CompilerParams(
                           use_tc_tiling_on_sc=False))
    def k(*refs):
        if with_counts:
            (table3, src4, dst4, zf, zc, ones_in, outf3, outc,
             srcv, dstv, rowb, gsem, ssem, accf, sem, onesv, accc) = refs
        else:
            (table3, src4, dst4, zf, outf3,
             srcv, dstv, rowb, gsem, ssem, accf, sem) = refs
        c = lax.axis_index("c")
        s = lax.axis_index("s")

        pltpu.sync_copy(zf.at[pl.ds(s * R0, R0)], accf.at[pl.ds(s * R0, R0)])
        if with_counts:
            pltpu.sync_copy(zc.at[pl.ds(s * R0, R0)],
                            accc.at[pl.ds(s * R0, R0)])
            pltpu.sync_copy(ones_in, onesv)

        @pl.when(s == NS - 1)
        def _():
            pltpu.sync_copy(zf.at[pl.ds(NS * R0, TAIL)],
                            accf.at[pl.ds(NS * R0, TAIL)])
            if with_counts:
                pltpu.sync_copy(zc.at[pl.ds(NS * R0, TAIL)],
                                accc.at[pl.ds(NS * R0, TAIL)])

        plsc.subcore_barrier()

        def blk(b, carry):
            pltpu.sync_copy(src4.at[s, b], srcv)
            pltpu.sync_copy(dst4.at[s, b], dstv)

            def start_g(q):
                return pltpu.async_copy(table3.at[c].at[srcv.at[q]],
                                        rowb[q % nbuf], gsem[q % nbuf])

            gd = [None] * NB
            for q in range(2):
                gd[q] = start_g(q)
            for j in range(NB):
                gd[j].wait()
                if j + 2 < NB:
                    gd[j + 2] = start_g(j + 2)
                pltpu.sync_copy(rowb[j % nbuf], accf.at[dstv.at[j]],
                                add=True)
                if with_counts:
                    @pl.when(c == j % 2)
                    def _():
                        pltpu.sync_copy(onesv, accc.at[dstv.at[j]],
                                        add=True)
            return carry

        lax.fori_loop(0, nblk, blk, 0)
        plsc.subcore_barrier()

        pltpu.sync_copy(accf.at[pl.ds(s * R0, R0)],
                        outf3.at[c].at[pl.ds(s * R0, R0)])
        if with_counts:
            pltpu.sync_copy(accc.at[pl.ds(s * R0, R0)],
                            outc.at[pl.ds(c * N + s * R0, R0)])

        @pl.when(s == NS - 1)
        def _():
            pltpu.sync_copy(accf.at[pl.ds(NS * R0, TAIL)],
                            outf3.at[c].at[pl.ds(NS * R0, TAIL)])
            if with_counts:
                pltpu.sync_copy(accc.at[pl.ds(NS * R0, TAIL)],
                                outc.at[pl.ds(c * N + NS * R0, TAIL)])

    return k


def _tc1_body(a3, c0, c1, xr, wl1a, wl1b, wr1, b1r, wl2a, wl2b, wr2, b2r,
              g3_ref, r2_ref):
    cnt = c0[...] + c1[...]
    rinv = 1.0 / jnp.maximum(cnt[:, 0:1], 1.0)
    h = jnp.dot(a3[0] * rinv, wl1a[...], preferred_element_type=jnp.float32)
    h += jnp.dot(a3[1] * rinv, wl1b[...], preferred_element_type=jnp.float32)
    h += jnp.dot(xr[...], wr1[...], preferred_element_type=jnp.float32)
    h = jnp.maximum(h + b1r[...], 0.0)
    g3_ref[0] = jnp.dot(h, wl2a[...], preferred_element_type=jnp.float32)
    g3_ref[1] = jnp.dot(h, wl2b[...], preferred_element_type=jnp.float32)
    r2_ref[...] = (jnp.dot(h, wr2[...], preferred_element_type=jnp.float32)
                   + b2r[...])


def _tc2_body(gs3, c0, c1, r2r, out_ref):
    cnt = c0[...] + c1[...]
    rinv = 1.0 / jnp.maximum(cnt[:, 0:1], 1.0)
    out_ref[:, 0:HD] = gs3[0] * rinv + r2r[:, 0:HD]
    out_ref[:, HD:D] = gs3[1] * rinv + r2r[:, HD:D]


def _half_spec():
    return pl.BlockSpec((2, BL, HD), lambda i: (0, i, 0))


def _cnt_spec(half):
    off = half * (N // BL)
    return pl.BlockSpec((BL, CW), lambda i, o=off: (i + o, 0))


def _full_spec(shape):
    n = len(shape)
    return pl.BlockSpec(shape, lambda i: (0,) * n)


def _row_spec():
    return pl.BlockSpec((BL, D), lambda i: (i, 0))


def kernel(x, edge_index, W_l1, W_r1, b1, W_l2, W_r2, b2):
    n_edges = edge_index.shape[1]
    nch = n_edges // NS // CHUNK
    nblk = nch // NB
    src4 = edge_index[0].reshape(NS, nblk, NB, CHUNK)
    dst4 = edge_index[1].reshape(NS, nblk, NB, CHUNK)
    x3 = x.reshape(N, 2, HD).transpose(1, 0, 2)
    zf = jnp.zeros((N, HD), jnp.float32)
    zc = jnp.zeros((N, CW), jnp.float32)
    ones_in = jnp.ones((CHUNK, CW), jnp.float32)
    b1r = b1.reshape(1, D)
    b2r = b2.reshape(1, D)
    wl1a, wl1b = W_l1[:HD], W_l1[HD:]
    wl2a, wl2b = W_l2[:, :HD], W_l2[:, HD:]

    agg3, cnts = _make_sc_agg(n_edges, True)(x3, src4, dst4, zf, zc, ones_in)

    grid = (N // BL,)
    g3, r2 = pl.pallas_call(
        _tc1_body,
        grid=grid,
        in_specs=[
            _half_spec(), _cnt_spec(0), _cnt_spec(1),
            _row_spec(),
            _full_spec((HD, D)), _full_spec((HD, D)), _full_spec((D, D)),
            _full_spec((1, D)),
            _full_spec((D, HD)), _full_spec((D, HD)), _full_spec((D, D)),
            _full_spec((1, D)),
        ],
        out_specs=[_half_spec(), _row_spec()],
        out_shape=[jax.ShapeDtypeStruct((2, N, HD), jnp.float32),
                   jax.ShapeDtypeStruct((N, D), jnp.float32)],
    )(agg3, cnts, cnts, x, wl1a, wl1b, W_r1, b1r, wl2a, wl2b, W_r2, b2r)

    gs3 = _make_sc_agg(n_edges, False)(g3, src4, dst4, zf)

    out = pl.pallas_call(
        _tc2_body,
        grid=grid,
        in_specs=[_half_spec(), _cnt_spec(0), _cnt_spec(1), _row_spec()],
        out_specs=_row_spec(),
        out_shape=jax.ShapeDtypeStruct((N, D), jnp.float32),
    )(gs3, cnts, cnts, r2)

    return out


# speedup vs baseline: 10.3150x; 1.0801x over previous
import functools

import jax
import jax.numpy as jnp
from jax import lax
from jax.experimental import pallas as pl
from jax.experimental.pallas import tpu as pltpu
from jax.experimental.pallas import tpu_sc as plsc

N = 10000
D = 128
HD = D // 2
CW = 16
NC, NS = 2, 16
CHUNK = 250
NB = 16
BL = 1000
R0 = (N // NS) // 8 * 8
TAIL = N - NS * R0


def _sc_mesh():
    return plsc.VectorSubcoreMesh(
        core_axis_name="c", subcore_axis_name="s",
        num_cores=NC, num_subcores=NS)


def _make_sc_agg(n_edges, with_counts):
    ept = n_edges // NS
    nch = ept // CHUNK
    nblk = nch // NB
    nbuf = 4

    out_type = [jax.ShapeDtypeStruct((NC, N, HD), jnp.float32)]
    scratch = [
        pltpu.VMEM((NB, CHUNK), jnp.int32),
        pltpu.VMEM((NB, CHUNK), jnp.int32),
        [pltpu.VMEM((CHUNK, HD), jnp.float32) for _ in range(nbuf)],
        [pltpu.SemaphoreType.DMA for _ in range(nbuf)],
        [pltpu.SemaphoreType.DMA for _ in range(nbuf)],
        pltpu.VMEM_SHARED((N, HD), jnp.float32),
        pltpu.SemaphoreType.DMA,
    ]
    if with_counts:
        out_type.append(jax.ShapeDtypeStruct((NC * N, CW), jnp.float32))
        scratch += [
            pltpu.VMEM((CHUNK, CW), jnp.float32),
            pltpu.VMEM_SHARED((N, CW), jnp.float32),
        ]

    @functools.partial(pl.kernel,
                       out_type=tuple(out_type) if with_counts else out_type[0],
                       mesh=_sc_mesh(), scratch_types=scratch,
                       compiler_params=pltpu.CompilerParams(
                           use_tc_tiling_on_sc=False))
    def k(*refs):
        if with_counts:
            (table3, src4, dst4, zf, zc, ones_in, outf3, outc,
             srcv, dstv, rowb, gsem, ssem, accf, sem, onesv, accc) = refs
        else:
            (table3, src4, dst4, zf, outf3,
             srcv, dstv, rowb, gsem, ssem, accf, sem) = refs
        c = lax.axis_index("c")
        s = lax.axis_index("s")

        pltpu.sync_copy(zf.at[pl.ds(s * R0, R0)], accf.at[pl.ds(s * R0, R0)])
        if with_counts:
            pltpu.sync_copy(zc.at[pl.ds(s * R0, R0)],
                            accc.at[pl.ds(s * R0, R0)])
            pltpu.sync_copy(ones_in, onesv)

        @pl.when(s == NS - 1)
        def _():
            pltpu.sync_copy(zf.at[pl.ds(NS * R0, TAIL)],
                            accf.at[pl.ds(NS * R0, TAIL)])
            if with_counts:
                pltpu.sync_copy(zc.at[pl.ds(NS * R0, TAIL)],
                                accc.at[pl.ds(NS * R0, TAIL)])

        plsc.subcore_barrier()

        def blk(b, carry):
            pltpu.sync_copy(src4.at[s, b], srcv)
            pltpu.sync_copy(dst4.at[s, b], dstv)

            def start_g(q):
                return pltpu.async_copy(table3.at[c].at[srcv.at[q]],
                                        rowb[q % nbuf], gsem[q % nbuf])

            gd = [None] * NB
            for q in range(2):
                gd[q] = start_g(q)
            for j in range(NB):
                gd[j].wait()
                if j + 2 < NB:
                    gd[j + 2] = start_g(j + 2)
                pltpu.sync_copy(rowb[j % nbuf], accf.at[dstv.at[j]],
                                add=True)
                if with_counts:
                    @pl.when(c == j % 2)
                    def _():
                        pltpu.sync_copy(onesv, accc.at[dstv.at[j]],
                                        add=True)
            return carry

        lax.fori_loop(0, nblk, blk, 0)
        plsc.subcore_barrier()

        pltpu.sync_copy(accf.at[pl.ds(s * R0, R0)],
                        outf3.at[c].at[pl.ds(s * R0, R0)])
        if with_counts:
            pltpu.sync_copy(accc.at[pl.ds(s * R0, R0)],
                            outc.at[pl.ds(c * N + s * R0, R0)])

        @pl.when(s == NS - 1)
        def _():
            pltpu.sync_copy(accf.at[pl.ds(NS * R0, TAIL)],
                            outf3.at[c].at[pl.ds(NS * R0, TAIL)])
            if with_counts:
                pltpu.sync_copy(accc.at[pl.ds(NS * R0, TAIL)],
                                outc.at[pl.ds(c * N + NS * R0, TAIL)])

    return k


def _tc1_body(a3, c0, c1, xr, wl1a, wl1b, wr1, b1r, wl2a, wl2b, wr2, b2r,
              g3_ref, r2_ref):
    cnt = c0[...] + c1[...]
    rinv = 1.0 / jnp.maximum(cnt[:, 0:1], 1.0)
    h = jnp.dot(a3[0] * rinv, wl1a[...], preferred_element_type=jnp.float32)
    h += jnp.dot(a3[1] * rinv, wl1b[...], preferred_element_type=jnp.float32)
    h += jnp.dot(xr[...], wr1[...], preferred_element_type=jnp.float32)
    h = jnp.maximum(h + b1r[...], 0.0)
    g3_ref[0] = jnp.dot(h, wl2a[...], preferred_element_type=jnp.float32)
    g3_ref[1] = jnp.dot(h, wl2b[...], preferred_element_type=jnp.float32)
    r2_ref[...] = (jnp.dot(h, wr2[...], preferred_element_type=jnp.float32)
                   + b2r[...])


def _tc2_body(gs3, c0, c1, r2r, out_ref):
    cnt = c0[...] + c1[...]
    rinv = 1.0 / jnp.maximum(cnt[:, 0:1], 1.0)
    out_ref[:, 0:HD] = gs3[0] * rinv + r2r[:, 0:HD]
    out_ref[:, HD:D] = gs3[1] * rinv + r2r[:, HD:D]


def _half_spec():
    return pl.BlockSpec((2, BL, HD), lambda i: (0, i, 0))


def _cnt_spec(half):
    off = half * (N // BL)
    return pl.BlockSpec((BL, CW), lambda i, o=off: (i + o, 0))


def _full_spec(shape):
    n = len(shape)
    return pl.BlockSpec(shape, lambda i: (0,) * n)


def _row_spec():
    return pl.BlockSpec((BL, D), lambda i: (i, 0))


def kernel(x, edge_index, W_l1, W_r1, b1, W_l2, W_r2, b2):
    n_edges = edge_index.shape[1]
    nch = n_edges // NS // CHUNK
    nblk = nch // NB
    src4 = edge_index[0].reshape(NS, nblk, NB, CHUNK)
    dst4 = edge_index[1].reshape(NS, nblk, NB, CHUNK)
    x3 = x.reshape(N, 2, HD).transpose(1, 0, 2)
    zf = jnp.zeros((N, HD), jnp.float32)
    zc = jnp.zeros((N, CW), jnp.float32)
    ones_in = jnp.ones((CHUNK, CW), jnp.float32)
    b1r = b1.reshape(1, D)
    b2r = b2.reshape(1, D)
    wl1a, wl1b = W_l1[:HD], W_l1[HD:]
    wl2a, wl2b = W_l2[:, :HD], W_l2[:, HD:]

    agg3, cnts = _make_sc_agg(n_edges, True)(x3, src4, dst4, zf, zc, ones_in)

    grid = (N // BL,)
    g3, r2 = pl.pallas_call(
        _tc1_body,
        grid=grid,
        in_specs=[
            _half_spec(), _cnt_spec(0), _cnt_spec(1),
            _row_spec(),
            _full_spec((HD, D)), _full_spec((HD, D)), _full_spec((D, D)),
            _full_spec((1, D)),
            _full_spec((D, HD)), _full_spec((D, HD)), _full_spec((D, D)),
            _full_spec((1, D)),
        ],
        out_specs=[_half_spec(), _row_spec()],
        out_shape=[jax.ShapeDtypeStruct((2, N, HD), jnp.float32),
                   jax.ShapeDtypeStruct((N, D), jnp.float32)],
    )(agg3, cnts, cnts, x, wl1a, wl1b, W_r1, b1r, wl2a, wl2b, W_r2, b2r)

    gs3 = _make_sc_agg(n_edges, False)(g3, src4, dst4, zf)

    out = pl.pallas_call(
        _tc2_body,
        grid=grid,
        in_specs=[_half_spec(), _cnt_spec(0), _cnt_spec(1), _row_spec()],
        out_specs=_row_spec(),
        out_shape=jax.ShapeDtypeStruct((N, D), jnp.float32),
    )(gs3, cnts, cnts, r2)

    return out
